# baseline (speedup 1.0000x reference)
import functools

import jax
import jax.numpy as jnp
import numpy as np
from jax.experimental import pallas as pl
from jax.experimental.pallas import tpu as pltpu

F32 = jnp.float32
BF16 = jnp.bfloat16


def _cp(n_par):
    return pltpu.CompilerParams(
        dimension_semantics=("parallel",) * n_par)


def _mm(a, b, trans_b=False):
    dims = (((1,), (1 if trans_b else 0,)), ((), ()))
    return jax.lax.dot_general(a, b, dims,
                               precision=jax.lax.Precision.DEFAULT,
                               preferred_element_type=F32)


def _expand_kernel(x_ref, s_ref, out_ref):
    x = x_ref[0]
    out_ref[0, :, 18:24, :] = jnp.zeros((226, 6, 221), BF16)
    for kw in range(6):
        r = _mm(x, s_ref[kw]).astype(BF16)
        for ci in range(3):
            out_ref[0, :, ci * 6 + kw, :] = r[ci * 226:(ci + 1) * 226]


def _expand(xpflat, shifts):
    return pl.pallas_call(
        _expand_kernel,
        grid=(8,),
        in_specs=[
            pl.BlockSpec((1, 678, 226), lambda b: (b, 0, 0)),
            pl.BlockSpec((6, 226, 221), lambda b: (0, 0, 0)),
        ],
        out_specs=pl.BlockSpec((1, 226, 24, 221), lambda b: (b, 0, 0, 0)),
        out_shape=jax.ShapeDtypeStruct((8, 226, 24, 221), BF16),
        compiler_params=_cp(1),
    )(xpflat, shifts)


_ROWS0 = 55


def _conv0_kernel(xw_ref, w_ref, sel_ref, out_ref):
    g = pl.program_id(1)
    w = w_ref[...]
    pooled = []
    for li in range(_ROWS0):
        acc = None
        for r in range(4):
            h = (g * _ROWS0 + li) * 4 + r
            a = jnp.concatenate([xw_ref[0, h + kh] for kh in range(6)],
                                axis=0)
            c = _mm(w, a)
            acc = c if acc is None else jnp.maximum(acc, c)
        acc = jnp.maximum(acc, 0.0)
        m = jnp.maximum(jnp.maximum(acc[:, 0:217], acc[:, 1:218]),
                        jnp.maximum(acc[:, 2:219], acc[:, 3:220]))
        pooled.append(m.astype(BF16))
    p = _mm(jnp.concatenate(pooled, axis=0), sel_ref[...])
    pt = jnp.swapaxes(p, 0, 1).astype(BF16)
    for li in range(_ROWS0):
        out_ref[0, li] = pt[:, li * 64:(li + 1) * 64]


def _conv0(xw, w144, sel):
    return pl.pallas_call(
        _conv0_kernel,
        grid=(8, 55 // _ROWS0),
        in_specs=[
            pl.BlockSpec((1, 226, 24, 221), lambda b, g: (b, 0, 0, 0)),
            pl.BlockSpec((64, 144), lambda b, g: (0, 0)),
            pl.BlockSpec((217, 55), lambda b, g: (0, 0)),
        ],
        out_specs=pl.BlockSpec((1, _ROWS0, 55, 64), lambda b, g: (b, g, 0, 0)),
        out_shape=jax.ShapeDtypeStruct((8, 55, 55, 64), BF16),
        compiler_params=_cp(2),
    )(xw, w144, sel)


def _convpool_kernel(x_ref, w_ref, se_ref, out_ref, *, rows, wc, co, out_t):
    g = pl.program_id(1)
    w = w_ref[...]

    pieces = {}
    for dh in range(2 * rows + 5):
        for kw in range(6):
            pieces[(dh, kw)] = x_ref[0, 2 * g * rows + dh, kw:kw + wc, :]

    def big_a(parity):
        parts = []
        for li in range(rows):
            dh = 2 * li + parity
            parts.append(jnp.concatenate(
                [pieces[(dh + kh, kw)]
                 for kh in range(6) for kw in range(6)], axis=1))
        return jnp.concatenate(parts, axis=0)

    m = jnp.maximum(_mm(big_a(0), w), _mm(big_a(1), w))
    m = jnp.maximum(m, 0.0)
    mshift = jnp.concatenate([m[1:], m[:1]], axis=0)
    mm2 = jnp.maximum(m, mshift).astype(BF16)
    p = _mm(se_ref[...], mm2)
    if out_t:
        out_ref[0] = jnp.swapaxes(p, 0, 1)
    else:
        out_ref[0, 0] = p.astype(BF16)


def _convpool(xnhwc, wr, se, hp, wc, rows, gsteps, co, out_t):
    nsel = se.shape[0]
    if out_t:
        out_spec = pl.BlockSpec((1, co, nsel), lambda b, g: (b, 0, 0))
        out_shape = jax.ShapeDtypeStruct((8, co, nsel), F32)
    else:
        out_spec = pl.BlockSpec((1, 1, nsel, co), lambda b, g: (b, g, 0, 0))
        out_shape = jax.ShapeDtypeStruct((8, gsteps, nsel, co), BF16)
    ci = xnhwc.shape[3]
    return pl.pallas_call(
        functools.partial(_convpool_kernel, rows=rows, wc=wc, co=co,
                          out_t=out_t),
        grid=(8, gsteps),
        in_specs=[
            pl.BlockSpec((1, hp, hp, ci), lambda b, g: (b, 0, 0, 0)),
            pl.BlockSpec(wr.shape, lambda b, g: (0, 0)),
            pl.BlockSpec(se.shape, lambda b, g: (0, 0)),
        ],
        out_specs=out_spec,
        out_shape=out_shape,
        compiler_params=_cp(2),
    )(xnhwc, wr, se)


def _fc1_kernel(x_ref, w_ref, b_ref, out_ref, *, nsteps):
    k = pl.program_id(0)
    part = _mm(x_ref[...], w_ref[...], trans_b=True)

    @pl.when(k == 0)
    def _():
        out_ref[...] = part

    @pl.when(k > 0)
    def _():
        out_ref[...] = out_ref[...] + part

    @pl.when(k == nsteps - 1)
    def _():
        h = jnp.maximum(out_ref[...] + b_ref[...], 0.0)
        n = jnp.maximum(jnp.sqrt(jnp.sum(h * h, axis=1, keepdims=True)),
                        1e-12)
        out_ref[...] = h / n


def _fc1(x, w, b2d):
    nsteps = 11
    kb = 30976 // nsteps
    return pl.pallas_call(
        functools.partial(_fc1_kernel, nsteps=nsteps),
        grid=(nsteps,),
        in_specs=[
            pl.BlockSpec((8, kb), lambda k: (0, k)),
            pl.BlockSpec((1024, kb), lambda k: (0, k)),
            pl.BlockSpec((1, 1024), lambda k: (0, 0)),
        ],
        out_specs=pl.BlockSpec((8, 1024), lambda k: (0, 0)),
        out_shape=jax.ShapeDtypeStruct((8, 1024), F32),
    )(x, w, b2d)


def _fc2_kernel(x_ref, w_ref, b_ref, out_ref):
    out_ref[...] = _mm(x_ref[...], w_ref[...], trans_b=True) + b_ref[...]


def _fc2(x, w, b2d):
    nb = 2048
    grid = (16500 + nb - 1) // nb
    return pl.pallas_call(
        _fc2_kernel,
        grid=(grid,),
        in_specs=[
            pl.BlockSpec((8, 1024), lambda n: (0, 0)),
            pl.BlockSpec((nb, 1024), lambda n: (n, 0)),
            pl.BlockSpec((1, nb), lambda n: (0, n)),
        ],
        out_specs=pl.BlockSpec((8, nb), lambda n: (0, n)),
        out_shape=jax.ShapeDtypeStruct((8, 16500), F32),
        compiler_params=_cp(1),
    )(x, w, b2d)


def _np_consts():
    i = np.arange(226)[:, None]
    wv = np.arange(221)[None, :]
    shifts = np.stack([(i == wv + kw) for kw in range(6)], 0)

    i = np.arange(217)[:, None]
    j = np.arange(55)[None, :]
    sel0 = (i == 4 * j)

    def left_sel(rows, wc, wo):
        q = np.arange(rows * wc)
        out_rows = [q == li * wc + 2 * j
                    for li in range(rows) for j in range(wo)]
        return np.stack(out_rows, 0)
    se1 = left_sel(26, 52, 26)
    se2 = left_sel(11, 23, 11)
    as_bf = lambda a: jnp.asarray(a.astype(np.float32), dtype=BF16)
    return as_bf(shifts), as_bf(sel0), as_bf(se1), as_bf(se2)


def kernel(input_, conv_w0, conv_w1, conv_w2, fc1_w, fc1_b, fc2_w, fc2_b):
    shifts, sel0, se1, se2 = _np_consts()

    xpf = jnp.pad(input_, ((0, 0), (0, 0), (1, 1), (1, 1))).astype(BF16)
    xw = _expand(xpf.reshape(8, 678, 226), shifts)
    wt = conv_w0.transpose(2, 1, 3, 0).reshape(6, 18, 64)
    wt = jnp.pad(wt, ((0, 0), (0, 6), (0, 0))).reshape(144, 64)
    w144 = (wt.T * (1.0 / 255.0)).astype(BF16)
    p0 = _conv0(xw, w144, sel0)

    xp1 = jnp.pad(p0, ((0, 0), (1, 1), (1, 1), (0, 0)))
    w1r = conv_w1.transpose(2, 3, 1, 0).reshape(2304, 128).astype(BF16)
    p1 = _convpool(xp1, w1r, se1, 57, 52, 26, 1, 128, False)

    xp2 = jnp.pad(p1.reshape(8, 26, 26, 128),
                  ((0, 0), (1, 1), (1, 1), (0, 0)))
    w2r = conv_w2.transpose(2, 3, 1, 0).reshape(4608, 256).astype(BF16)
    p2 = _convpool(xp2, w2r, se2, 28, 23, 11, 1, 256, True)

    flat = p2.reshape(8, 30976)
    xn = _fc1(flat, fc1_w, fc1_b.reshape(1, 1024))
    out = _fc2(xn, fc2_w, fc2_b.reshape(1, 16500))
    return out.reshape(8, 1100, 15)


# speedup vs baseline: 1.0171x; 1.0171x over previous
import functools

import jax
import jax.numpy as jnp
import numpy as np
from jax.experimental import pallas as pl
from jax.experimental.pallas import tpu as pltpu

F32 = jnp.float32
BF16 = jnp.bfloat16


def _cp(n_par):
    return pltpu.CompilerParams(
        dimension_semantics=("parallel",) * n_par)


def _mm(a, b, trans_b=False):
    dims = (((1,), (1 if trans_b else 0,)), ((), ()))
    return jax.lax.dot_general(a, b, dims,
                               precision=jax.lax.Precision.DEFAULT,
                               preferred_element_type=F32)


def _expand_kernel(x_ref, s_ref, out_ref):
    x = x_ref[0]
    out_ref[0, :, 18:24, :] = jnp.zeros((226, 6, 221), BF16)
    for kw in range(6):
        r = _mm(x, s_ref[kw]).astype(BF16)
        for ci in range(3):
            out_ref[0, :, ci * 6 + kw, :] = r[ci * 226:(ci + 1) * 226]


def _expand(xpflat, shifts):
    return pl.pallas_call(
        _expand_kernel,
        grid=(8,),
        in_specs=[
            pl.BlockSpec((1, 678, 226), lambda b: (b, 0, 0)),
            pl.BlockSpec((6, 226, 221), lambda b: (0, 0, 0)),
        ],
        out_specs=pl.BlockSpec((1, 226, 24, 221), lambda b: (b, 0, 0, 0)),
        out_shape=jax.ShapeDtypeStruct((8, 226, 24, 221), BF16),
        compiler_params=_cp(1),
    )(xpflat, shifts)


_ROWS0 = 55


def _conv0_kernel(xw_ref, w_ref, sel_ref, out_ref):
    g = pl.program_id(1)
    w = w_ref[...]
    pooled = []
    for li in range(_ROWS0):
        acc = None
        for r in range(4):
            h = (g * _ROWS0 + li) * 4 + r
            a = jnp.concatenate([xw_ref[0, h + kh] for kh in range(6)],
                                axis=0)
            c = _mm(w, a)
            acc = c if acc is None else jnp.maximum(acc, c)
        acc = jnp.maximum(acc, 0.0)
        m = jnp.maximum(jnp.maximum(acc[:, 0:217], acc[:, 1:218]),
                        jnp.maximum(acc[:, 2:219], acc[:, 3:220]))
        pooled.append(m.astype(BF16))
    p = _mm(jnp.concatenate(pooled, axis=0), sel_ref[...])
    pt = jnp.swapaxes(p, 0, 1).astype(BF16)
    for li in range(_ROWS0):
        out_ref[0, li] = pt[:, li * 64:(li + 1) * 64]


def _conv0(xw, w144, sel):
    return pl.pallas_call(
        _conv0_kernel,
        grid=(8, 55 // _ROWS0),
        in_specs=[
            pl.BlockSpec((1, 226, 24, 221), lambda b, g: (b, 0, 0, 0)),
            pl.BlockSpec((64, 144), lambda b, g: (0, 0)),
            pl.BlockSpec((217, 55), lambda b, g: (0, 0)),
        ],
        out_specs=pl.BlockSpec((1, _ROWS0, 55, 64), lambda b, g: (b, g, 0, 0)),
        out_shape=jax.ShapeDtypeStruct((8, 55, 55, 64), BF16),
        compiler_params=_cp(2),
    )(xw, w144, sel)


def _convpool_kernel(x_ref, w_ref, se_ref, out_ref, *, rows, wc, co, out_t):
    g = pl.program_id(1)
    w = w_ref[...]

    pieces = {}
    for dh in range(2 * rows + 5):
        for kw in range(6):
            pieces[(dh, kw)] = x_ref[0, 2 * g * rows + dh, kw:kw + wc, :]

    def big_a(parity):
        parts = []
        for li in range(rows):
            dh = 2 * li + parity
            parts.append(jnp.concatenate(
                [pieces[(dh + kh, kw)]
                 for kh in range(6) for kw in range(6)], axis=1))
        return jnp.concatenate(parts, axis=0)

    m = jnp.maximum(_mm(big_a(0), w), _mm(big_a(1), w))
    m = jnp.maximum(m, 0.0)
    mshift = jnp.concatenate([m[1:], m[:1]], axis=0)
    mm2 = jnp.maximum(m, mshift).astype(BF16)
    p = _mm(se_ref[...], mm2)
    if out_t:
        out_ref[0] = jnp.swapaxes(p, 0, 1)
    else:
        out_ref[0, 0] = p.astype(BF16)


def _convpool(xnhwc, wr, se, hp, wc, rows, gsteps, co, out_t):
    nsel = se.shape[0]
    if out_t:
        out_spec = pl.BlockSpec((1, co, nsel), lambda b, g: (b, 0, 0))
        out_shape = jax.ShapeDtypeStruct((8, co, nsel), F32)
    else:
        out_spec = pl.BlockSpec((1, 1, nsel, co), lambda b, g: (b, g, 0, 0))
        out_shape = jax.ShapeDtypeStruct((8, gsteps, nsel, co), BF16)
    ci = xnhwc.shape[3]
    return pl.pallas_call(
        functools.partial(_convpool_kernel, rows=rows, wc=wc, co=co,
                          out_t=out_t),
        grid=(8, gsteps),
        in_specs=[
            pl.BlockSpec((1, hp, hp, ci), lambda b, g: (b, 0, 0, 0)),
            pl.BlockSpec(wr.shape, lambda b, g: (0, 0)),
            pl.BlockSpec(se.shape, lambda b, g: (0, 0)),
        ],
        out_specs=out_spec,
        out_shape=out_shape,
        compiler_params=_cp(2),
    )(xnhwc, wr, se)


def _fc1_kernel(x_ref, w_ref, b_ref, out_ref, *, nsteps):
    k = pl.program_id(0)
    part = _mm(x_ref[...], w_ref[...], trans_b=True)

    @pl.when(k == 0)
    def _():
        out_ref[...] = part

    @pl.when(k > 0)
    def _():
        out_ref[...] = out_ref[...] + part

    @pl.when(k == nsteps - 1)
    def _():
        h = jnp.maximum(out_ref[...] + b_ref[...], 0.0)
        n = jnp.maximum(jnp.sqrt(jnp.sum(h * h, axis=1, keepdims=True)),
                        1e-12)
        out_ref[...] = h / n


def _fc1(x, w, b2d):
    nsteps = 11
    kb = 30976 // nsteps
    return pl.pallas_call(
        functools.partial(_fc1_kernel, nsteps=nsteps),
        grid=(nsteps,),
        in_specs=[
            pl.BlockSpec((8, kb), lambda k: (0, k)),
            pl.BlockSpec((1024, kb), lambda k: (0, k)),
            pl.BlockSpec((1, 1024), lambda k: (0, 0)),
        ],
        out_specs=pl.BlockSpec((8, 1024), lambda k: (0, 0)),
        out_shape=jax.ShapeDtypeStruct((8, 1024), F32),
    )(x, w, b2d)


def _fc2_kernel(x_ref, w_ref, b_ref, out_ref):
    out_ref[...] = _mm(x_ref[...], w_ref[...], trans_b=True) + b_ref[...]


def _fc2(x, w, b2d):
    nb = 2048
    grid = (16500 + nb - 1) // nb
    return pl.pallas_call(
        _fc2_kernel,
        grid=(grid,),
        in_specs=[
            pl.BlockSpec((8, 1024), lambda n: (0, 0)),
            pl.BlockSpec((nb, 1024), lambda n: (n, 0)),
            pl.BlockSpec((1, nb), lambda n: (0, n)),
        ],
        out_specs=pl.BlockSpec((8, nb), lambda n: (0, n)),
        out_shape=jax.ShapeDtypeStruct((8, 16500), F32),
        compiler_params=_cp(1),
    )(x, w, b2d)


def _np_consts():
    i = np.arange(226)[:, None]
    wv = np.arange(221)[None, :]
    shifts = np.stack([(i == wv + kw) for kw in range(6)], 0)

    i = np.arange(217)[:, None]
    j = np.arange(55)[None, :]
    sel0 = (i == 4 * j)

    def left_sel(rows, wc, wo):
        q = np.arange(rows * wc)
        out_rows = [q == li * wc + 2 * j
                    for li in range(rows) for j in range(wo)]
        return np.stack(out_rows, 0)
    se1 = left_sel(13, 52, 26)
    se2 = left_sel(11, 23, 11)
    as_bf = lambda a: jnp.asarray(a.astype(np.float32), dtype=BF16)
    return as_bf(shifts), as_bf(sel0), as_bf(se1), as_bf(se2)


def kernel(input_, conv_w0, conv_w1, conv_w2, fc1_w, fc1_b, fc2_w, fc2_b):
    shifts, sel0, se1, se2 = _np_consts()

    xpf = jnp.pad(input_, ((0, 0), (0, 0), (1, 1), (1, 1))).astype(BF16)
    xw = _expand(xpf.reshape(8, 678, 226), shifts)
    wt = conv_w0.transpose(2, 1, 3, 0).reshape(6, 18, 64)
    wt = jnp.pad(wt, ((0, 0), (0, 6), (0, 0))).reshape(144, 64)
    w144 = (wt.T * (1.0 / 255.0)).astype(BF16)
    p0 = _conv0(xw, w144, sel0)

    xp1 = jnp.pad(p0, ((0, 0), (1, 1), (1, 1), (0, 0)))
    w1r = conv_w1.transpose(2, 3, 1, 0).reshape(2304, 128).astype(BF16)
    p1 = _convpool(xp1, w1r, se1, 57, 52, 13, 2, 128, False)

    xp2 = jnp.pad(p1.reshape(8, 26, 26, 128),
                  ((0, 0), (1, 1), (1, 1), (0, 0)))
    w2r = conv_w2.transpose(2, 3, 1, 0).reshape(4608, 256).astype(BF16)
    p2 = _convpool(xp2, w2r, se2, 28, 23, 11, 1, 256, True)

    flat = p2.reshape(8, 30976)
    xn = _fc1(flat, fc1_w, fc1_b.reshape(1, 1024))
    out = _fc2(xn, fc2_w, fc2_b.reshape(1, 16500))
    return out.reshape(8, 1100, 15)
